# Initial kernel scaffold; baseline (speedup 1.0000x reference)
import jax
import jax.numpy as jnp
from jax.experimental import pallas as pl


def kernel(x, edge_index, adj0_vals, adj1_vals, W):
    raise NotImplementedError("write your pallas kernel here")


# speedup vs baseline: 4.1061x; 4.1061x over previous
import functools

import jax
import jax.numpy as jnp
from jax import lax
from jax.experimental import pallas as pl
from jax.experimental.pallas import tpu as pltpu
from jax.experimental.pallas import tpu_sc as plsc

N_NODES = 10000
D = 128
N_EDGES = 320000

NUM_CORES = 2
NUM_SUBCORES = 16
EDGES_PER_TILE = N_EDGES // NUM_SUBCORES
K = 80
NCHUNK = EDGES_PER_TILE // K
ROWS_PER_TILE = 624
ZROWS = 208


def _dense_body(x_ref, w_ref, a_ref, b_ref):
    h = jnp.dot(x_ref[...], w_ref[...], preferred_element_type=jnp.float32)
    var = jnp.maximum(h, 0.0)
    mean = jnp.where(h > 0.0, h, jnp.exp(h) - 1.0)
    att = jnp.exp(-var)
    a_ref[...] = mean * att
    b_ref[...] = var * att * att


_dense = pl.pallas_call(
    _dense_body,
    grid=(10,),
    in_specs=[
        pl.BlockSpec((1000, D), lambda i: (i, 0)),
        pl.BlockSpec((D, D), lambda i: (0, 0)),
    ],
    out_specs=[
        pl.BlockSpec((1000, D), lambda i: (i, 0)),
        pl.BlockSpec((1000, D), lambda i: (i, 0)),
    ],
    out_shape=[
        jax.ShapeDtypeStruct((N_NODES, D), jnp.float32),
        jax.ShapeDtypeStruct((N_NODES, D), jnp.float32),
    ],
)


def _bcast_lane(vec, l):
    return lax.gather(
        vec,
        jnp.full((16, 1), l, jnp.int32),
        lax.GatherDimensionNumbers(
            offset_dims=(), collapsed_slice_dims=(0,), start_index_map=(0,)),
        (1,),
        mode=lax.GatherScatterMode.PROMISE_IN_BOUNDS,
    )


def _spmm_body(a_hbm, b_hbm, rows_hbm, cols_hbm, v0_hbm, v1_hbm, mean_hbm, var_hbm,
               acc, zbuf, colbuf, rowbuf, valbuf_v, gath, sem):
    c = lax.axis_index("c")
    s = lax.axis_index("s")
    row0 = s * ROWS_PER_TILE

    def _zrow(i, carry):
        for j in range(8):
            zbuf[i, pl.ds(16 * j, 16)] = jnp.zeros((16,), jnp.float32)
        return carry

    lax.fori_loop(0, ZROWS, _zrow, None)
    for r in range(3):
        pltpu.sync_copy(zbuf, acc.at[pl.ds(row0 + r * ZROWS, ZROWS)])

    @pl.when(s == NUM_SUBCORES - 1)
    def _():
        pltpu.sync_copy(zbuf.at[pl.ds(0, 16)],
                        acc.at[pl.ds(NUM_SUBCORES * ROWS_PER_TILE, 16)])

    plsc.subcore_barrier()

    def _phase(dense_hbm, vals_hbm):
        ebase = s * EDGES_PER_TILE

        def _chunk(i, carry):
            base = pl.multiple_of(ebase + i * K, 8)
            pltpu.sync_copy(cols_hbm.at[pl.ds(base, K)], colbuf)
            pltpu.sync_copy(rows_hbm.at[pl.ds(base, K)], rowbuf)
            pltpu.sync_copy(vals_hbm.at[pl.ds(base, K)], valbuf_v)
            pltpu.async_copy(dense_hbm.at[colbuf], gath, sem).wait()

            def _scale(g, c2):
                vv = valbuf_v[pl.ds(g * 16, 16)]
                for l in range(16):
                    bl = _bcast_lane(vv, l)
                    e = g * 16 + l
                    for j in range(8):
                        gath[e, pl.ds(16 * j, 16)] = (
                            gath[e, pl.ds(16 * j, 16)] * bl)
                return c2

            lax.fori_loop(0, K // 16, _scale, None)
            pltpu.sync_copy(gath, acc.at[rowbuf], add=True)
            return carry

        lax.fori_loop(0, NCHUNK, _chunk, None)

    @pl.when(c == 0)
    def _():
        _phase(a_hbm, v0_hbm)

    @pl.when(c == 1)
    def _():
        _phase(b_hbm, v1_hbm)

    plsc.subcore_barrier()

    tail0 = NUM_SUBCORES * ROWS_PER_TILE

    @pl.when(c == 0)
    def _():
        pltpu.sync_copy(acc.at[pl.ds(row0, ROWS_PER_TILE)],
                        mean_hbm.at[pl.ds(row0, ROWS_PER_TILE)])

        @pl.when(s == NUM_SUBCORES - 1)
        def _():
            pltpu.sync_copy(acc.at[pl.ds(tail0, N_NODES - tail0)],
                            mean_hbm.at[pl.ds(tail0, N_NODES - tail0)])

    @pl.when(c == 1)
    def _():
        pltpu.sync_copy(acc.at[pl.ds(row0, ROWS_PER_TILE)],
                        var_hbm.at[pl.ds(row0, ROWS_PER_TILE)])

        @pl.when(s == NUM_SUBCORES - 1)
        def _():
            pltpu.sync_copy(acc.at[pl.ds(tail0, N_NODES - tail0)],
                            var_hbm.at[pl.ds(tail0, N_NODES - tail0)])


_spmm = pl.kernel(
    _spmm_body,
    out_type=(
        jax.ShapeDtypeStruct((N_NODES, D), jnp.float32),
        jax.ShapeDtypeStruct((N_NODES, D), jnp.float32),
    ),
    mesh=plsc.VectorSubcoreMesh(
        core_axis_name="c", subcore_axis_name="s",
        num_cores=NUM_CORES, num_subcores=NUM_SUBCORES,
    ),
    scratch_types=[
        pltpu.VMEM_SHARED((N_NODES, D), jnp.float32),
        pltpu.VMEM((ZROWS, D), jnp.float32),
        pltpu.VMEM((K,), jnp.int32),
        pltpu.VMEM((K,), jnp.int32),
        pltpu.VMEM((K,), jnp.float32),
        pltpu.VMEM((K, D), jnp.float32),
        pltpu.SemaphoreType.DMA,
    ],
)


@jax.jit
def kernel(x, edge_index, adj0_vals, adj1_vals, W):
    a, b = _dense(x, W)
    rows = edge_index[0]
    cols = edge_index[1]
    mean_out, var_out = _spmm(a, b, rows, cols, adj0_vals, adj1_vals)
    return (mean_out, var_out)


# speedup vs baseline: 6.2660x; 1.5260x over previous
import jax
import jax.numpy as jnp
from jax import lax
from jax.experimental import pallas as pl
from jax.experimental.pallas import tpu as pltpu
from jax.experimental.pallas import tpu_sc as plsc

N_NODES = 10000
D = 128
N_EDGES = 320000

NUM_CORES = 2
NUM_SUBCORES = 16
EDGES_PER_TILE = N_EDGES // NUM_SUBCORES
K = 64
NCHUNK = 316
EPT_PAD = NCHUNK * K
ROWS_PER_TILE = 624
ZROWS = 48


def _dense_body(x_ref, w_ref, a_ref, b_ref):
    h = jnp.dot(x_ref[...], w_ref[...], preferred_element_type=jnp.float32)
    var = jnp.maximum(h, 0.0)
    mean = jnp.where(h > 0.0, h, jnp.exp(h) - 1.0)
    att = jnp.exp(-var)
    a_ref[...] = mean * att
    b_ref[...] = var * att * att


_dense = pl.pallas_call(
    _dense_body,
    grid=(10,),
    in_specs=[
        pl.BlockSpec((1000, D), lambda i: (i, 0)),
        pl.BlockSpec((D, D), lambda i: (0, 0)),
    ],
    out_specs=[
        pl.BlockSpec((1000, D), lambda i: (i, 0)),
        pl.BlockSpec((1000, D), lambda i: (i, 0)),
    ],
    out_shape=[
        jax.ShapeDtypeStruct((N_NODES, D), jnp.float32),
        jax.ShapeDtypeStruct((N_NODES, D), jnp.float32),
    ],
)


def _bcast_lane(vec, l):
    return lax.gather(
        vec,
        jnp.full((16, 1), l, jnp.int32),
        lax.GatherDimensionNumbers(
            offset_dims=(), collapsed_slice_dims=(0,), start_index_map=(0,)),
        (1,),
        mode=lax.GatherScatterMode.PROMISE_IN_BOUNDS,
    )


def _spmm_body(a_hbm, b_hbm, p_hbm, v0_hbm, v1_hbm, mean_hbm, var_hbm,
               acc, zbuf, idxbuf, valbuf, gath0, gath1, scl0, scl1,
               sg0, sg1, ss0, ss1, si0, si1, si2, si3):
    c = lax.axis_index("c")
    s = lax.axis_index("s")
    row0 = s * ROWS_PER_TILE
    gath = (gath0, gath1)
    scl = (scl0, scl1)
    sg = (sg0, sg1)
    ss = (ss0, ss1)
    si = (si0, si1, si2, si3)

    def _zrow(i, carry):
        for j in range(8):
            zbuf[i, pl.ds(16 * j, 16)] = jnp.zeros((16,), jnp.float32)
        return carry

    lax.fori_loop(0, ZROWS, _zrow, None)
    for r in range(13):
        pltpu.sync_copy(zbuf, acc.at[pl.ds(row0 + r * ZROWS, ZROWS)])

    @pl.when(s == NUM_SUBCORES - 1)
    def _():
        pltpu.sync_copy(zbuf.at[pl.ds(0, 16)],
                        acc.at[pl.ds(NUM_SUBCORES * ROWS_PER_TILE, 16)])

    plsc.subcore_barrier()

    def _phase(dense_hbm, vals_hbm):
        dummy_g = dense_hbm.at[pl.ds(0, K)]
        dummy_i = p_hbm.at[s, 0]
        dummy_v = vals_hbm.at[s, 0]

        def _fetch_idx(j, slot):
            pltpu.async_copy(p_hbm.at[s, j], idxbuf.at[slot], si[slot])
            pltpu.async_copy(vals_hbm.at[s, j], valbuf.at[slot], si[slot])

        def _wait_idx(slot):
            pltpu.make_async_copy(dummy_i, idxbuf.at[slot], si[slot]).wait()
            pltpu.make_async_copy(dummy_v, valbuf.at[slot], si[slot]).wait()

        def _scale(slot, b2):
            def _grp(g, carry):
                vv = valbuf[slot, pl.ds(g * 16, 16)]
                for l in range(16):
                    bl = _bcast_lane(vv, l)
                    e = g * 16 + l
                    for jj in range(8):
                        scl[b2][e, pl.ds(16 * jj, 16)] = (
                            gath[b2][e, pl.ds(16 * jj, 16)] * bl)
                return carry

            lax.fori_loop(0, K // 16, _grp, None)

        def _sub(j, r):
            b2 = r % 2
            nb2 = 1 - b2
            s_cur = r % 4
            s_nxt = (r + 1) % 4
            s_pf = (r + 2) % 4

            pltpu.make_async_copy(dummy_g, gath[b2], sg[b2]).wait()

            @pl.when(j + 1 < NCHUNK)
            def _():
                _wait_idx(s_nxt)
                pltpu.async_copy(dense_hbm.at[idxbuf.at[s_nxt, 1]],
                                 gath[nb2], sg[nb2])

            @pl.when(j >= 2)
            def _():
                pltpu.make_async_copy(dummy_g, scl[b2], ss[b2]).wait()

            @pl.when(j + 2 < NCHUNK)
            def _():
                _fetch_idx(j + 2, s_pf)

            _scale(s_cur, b2)
            pltpu.async_copy(scl[b2], acc.at[idxbuf.at[s_cur, 0]],
                             ss[b2], add=True)

        _fetch_idx(0, 0)
        _fetch_idx(1, 1)
        _wait_idx(0)
        pltpu.async_copy(dense_hbm.at[idxbuf.at[0, 1]], gath0, sg0)

        def _quad(i4, carry):
            for r in range(4):
                _sub(4 * i4 + r, r)
            return carry

        lax.fori_loop(0, NCHUNK // 4, _quad, None)
        pltpu.make_async_copy(dummy_g, scl0, ss0).wait()
        pltpu.make_async_copy(dummy_g, scl1, ss1).wait()

    @pl.when(c == 0)
    def _():
        _phase(a_hbm, v0_hbm)

    @pl.when(c == 1)
    def _():
        _phase(b_hbm, v1_hbm)

    plsc.subcore_barrier()

    tail0 = NUM_SUBCORES * ROWS_PER_TILE

    @pl.when(c == 0)
    def _():
        pltpu.sync_copy(acc.at[pl.ds(row0, ROWS_PER_TILE)],
                        mean_hbm.at[pl.ds(row0, ROWS_PER_TILE)])

        @pl.when(s == NUM_SUBCORES - 1)
        def _():
            pltpu.sync_copy(acc.at[pl.ds(tail0, N_NODES - tail0)],
                            mean_hbm.at[pl.ds(tail0, N_NODES - tail0)])

    @pl.when(c == 1)
    def _():
        pltpu.sync_copy(acc.at[pl.ds(row0, ROWS_PER_TILE)],
                        var_hbm.at[pl.ds(row0, ROWS_PER_TILE)])

        @pl.when(s == NUM_SUBCORES - 1)
        def _():
            pltpu.sync_copy(acc.at[pl.ds(tail0, N_NODES - tail0)],
                            var_hbm.at[pl.ds(tail0, N_NODES - tail0)])


_spmm = pl.kernel(
    _spmm_body,
    out_type=(
        jax.ShapeDtypeStruct((N_NODES, D), jnp.float32),
        jax.ShapeDtypeStruct((N_NODES, D), jnp.float32),
    ),
    mesh=plsc.VectorSubcoreMesh(
        core_axis_name="c", subcore_axis_name="s",
        num_cores=NUM_CORES, num_subcores=NUM_SUBCORES,
    ),
    scratch_types=[
        pltpu.VMEM_SHARED((N_NODES, D), jnp.float32),
        pltpu.VMEM((ZROWS, D), jnp.float32),
        pltpu.VMEM((4, 2, K), jnp.int32),
        pltpu.VMEM((4, K), jnp.float32),
        pltpu.VMEM((K, D), jnp.float32),
        pltpu.VMEM((K, D), jnp.float32),
        pltpu.VMEM((K, D), jnp.float32),
        pltpu.VMEM((K, D), jnp.float32),
        pltpu.SemaphoreType.DMA,
        pltpu.SemaphoreType.DMA,
        pltpu.SemaphoreType.DMA,
        pltpu.SemaphoreType.DMA,
        pltpu.SemaphoreType.DMA,
        pltpu.SemaphoreType.DMA,
        pltpu.SemaphoreType.DMA,
        pltpu.SemaphoreType.DMA,
    ],
)


def _pad_tiles(arr):
    pad = EPT_PAD - EDGES_PER_TILE
    return jnp.pad(arr.reshape(NUM_SUBCORES, EDGES_PER_TILE),
                   ((0, 0), (0, pad)))


@jax.jit
def kernel(x, edge_index, adj0_vals, adj1_vals, W):
    a, b = _dense(x, W)
    r = _pad_tiles(edge_index[0])
    c = _pad_tiles(edge_index[1])
    p = jnp.stack([r, c], axis=1).reshape(
        NUM_SUBCORES, 2, NCHUNK, K).transpose(0, 2, 1, 3)
    v0 = _pad_tiles(adj0_vals).reshape(NUM_SUBCORES, NCHUNK, K)
    v1 = _pad_tiles(adj1_vals).reshape(NUM_SUBCORES, NCHUNK, K)
    mean_out, var_out = _spmm(a, b, p, v0, v1)
    return (mean_out, var_out)
